# baseline (speedup 1.0000x reference)
import functools

import jax
import jax.numpy as jnp
from jax import lax
from jax.experimental import pallas as pl
from jax.experimental.pallas import tpu as pltpu
from jax.experimental.pallas import tpu_sc as plsc

_N = 8192
_D = 256
_K = 512
_TB = 2048
_NBLK = _N // _TB

_NC = 2
_NS = 16
_NW = _NC * _NS
_BPW = _N // _NW
_CHUNK = 128
_CPW = _BPW // _CHUNK


def _argmin_body(x_ref, cb_ref, idx_ref, msum_ref):
    i = pl.program_id(0)
    x = x_ref[...]
    cb = cb_ref[...]
    x_sq = jnp.sum(x * x, axis=1, keepdims=True)
    x_sq_row = jnp.transpose(jnp.broadcast_to(x_sq, (_TB, 128)))[0:1, :]
    c_sq = jnp.sum(cb * cb, axis=1, keepdims=True)
    cross = lax.dot_general(
        cb, x, (((1,), (1,)), ((), ())),
        preferred_element_type=jnp.float32)
    dist_sq = jnp.maximum(x_sq_row + c_sq - 2.0 * cross, 0.0)
    dists = jnp.sqrt(dist_sq)
    minval = jnp.min(dists, axis=0, keepdims=True)
    row = lax.broadcasted_iota(jnp.int32, (_K, _TB), 0)
    idx = jnp.min(jnp.where(dists == minval, row, _K), axis=0)
    idx_ref[...] = idx.reshape(_TB // _CHUNK, _CHUNK)
    minsq = minval[0] * minval[0]

    @pl.when(i == 0)
    def _init():
        msum_ref[0, 0, :] = minsq

    @pl.when(i > 0)
    def _acc():
        msum_ref[0, 0, :] = msum_ref[0, 0, :] + minsq


def _tc_argmin(x2d, codebook):
    return pl.pallas_call(
        _argmin_body,
        grid=(_NBLK,),
        in_specs=[
            pl.BlockSpec((_TB, _D), lambda i: (i, 0)),
            pl.BlockSpec((_K, _D), lambda i: (0, 0)),
        ],
        out_specs=[
            pl.BlockSpec((_TB // _CHUNK, _CHUNK), lambda i: (i, 0)),
            pl.BlockSpec((1, 1, _TB), lambda i: (0, 0, 0)),
        ],
        out_shape=[
            jax.ShapeDtypeStruct((_N // _CHUNK, _CHUNK), jnp.int32),
            jax.ShapeDtypeStruct((1, 1, _TB), jnp.float32),
        ],
    )(x2d, codebook)


def _sc_gather(codebook, idx2d):
    mesh = plsc.VectorSubcoreMesh(
        core_axis_name="c", subcore_axis_name="s",
        num_cores=_NC, num_subcores=_NS)
    wpb = _TB // _BPW

    @functools.partial(
        pl.kernel,
        out_type=jax.ShapeDtypeStruct((_N, _D), jnp.float32),
        mesh=mesh,
        compiler_params=pltpu.CompilerParams(skip_device_barrier=True),
        scratch_types=[
            pltpu.VMEM((_CPW, _CHUNK), jnp.int32),
            pltpu.VMEM((_BPW, _D), jnp.float32),
            pltpu.SemaphoreType.DMA,
        ],
    )
    def gather_kernel(table_hbm, idx_hbm, out_q, idx_v, rows_v, sem):
        wid = lax.axis_index("s") * _NC + lax.axis_index("c")
        pltpu.sync_copy(idx_hbm.at[pl.ds(wid * _CPW, _CPW)], idx_v)
        copies = [
            pltpu.async_copy(table_hbm.at[idx_v.at[j]],
                             rows_v.at[pl.ds(j * _CHUNK, _CHUNK)], sem)
            for j in range(_CPW)
        ]
        for c in copies:
            c.wait()
        pltpu.sync_copy(rows_v, out_q.at[pl.ds(wid * _BPW, _BPW)])

    return gather_kernel(codebook, idx2d)


def kernel(x, codebook):
    B, T, D = x.shape
    x2d = x.reshape(B * T, D)
    idx2d, msum = _tc_argmin(x2d, codebook)
    quant2d = _sc_gather(codebook, idx2d)
    quantized = quant2d.reshape(B, T, D)
    indices = idx2d.reshape(B, T)
    loss = 2.0 * jnp.sum(msum[0, 0, :]) / jnp.float32(B * T * D)
    return (quantized, indices, loss)


# speedup vs baseline: 1.0020x; 1.0020x over previous
import functools

import jax
import jax.numpy as jnp
from jax import lax
from jax.experimental import pallas as pl
from jax.experimental.pallas import tpu as pltpu
from jax.experimental.pallas import tpu_sc as plsc

_N = 8192
_D = 256
_K = 512
_TB = 2048
_NBLK = _N // _TB

_NC = 2
_NS = 16
_NW = _NC * _NS
_BPW = _N // _NW
_CHUNK = 128
_CPW = _BPW // _CHUNK


def _argmin_body(x_ref, cb_ref, idx_ref, msum_ref):
    i = pl.program_id(0)
    x = x_ref[...]
    cb = cb_ref[...]
    x_sq = jnp.sum(x * x, axis=1, keepdims=True)
    x_sq_row = jnp.transpose(jnp.broadcast_to(x_sq, (_TB, 128)))[0:1, :]
    c_sq = jnp.sum(cb * cb, axis=1, keepdims=True)
    cross = lax.dot_general(
        cb, x, (((1,), (1,)), ((), ())),
        preferred_element_type=jnp.float32)
    dist_sq = jnp.maximum(x_sq_row + c_sq - 2.0 * cross, 0.0)
    dists = jnp.sqrt(dist_sq)
    minval = jnp.min(dists, axis=0, keepdims=True)
    row = lax.broadcasted_iota(jnp.int32, (_K, _TB), 0)
    idx = jnp.min(jnp.where(dists == minval, row, _K), axis=0)
    idx_ref[...] = idx.reshape(_TB // _CHUNK, _CHUNK)
    minsq = minval[0] * minval[0]

    @pl.when(i == 0)
    def _init():
        msum_ref[0, 0, :] = minsq

    @pl.when(i > 0)
    def _acc():
        msum_ref[0, 0, :] = msum_ref[0, 0, :] + minsq


def _tc_argmin(x2d, codebook):
    return pl.pallas_call(
        _argmin_body,
        grid=(_NBLK,),
        in_specs=[
            pl.BlockSpec((_TB, _D), lambda i: (i, 0)),
            pl.BlockSpec((_K, _D), lambda i: (0, 0)),
        ],
        out_specs=[
            pl.BlockSpec((_TB // _CHUNK, _CHUNK), lambda i: (i, 0)),
            pl.BlockSpec((1, 1, _TB), lambda i: (0, 0, 0)),
        ],
        out_shape=[
            jax.ShapeDtypeStruct((_N // _CHUNK, _CHUNK), jnp.int32),
            jax.ShapeDtypeStruct((1, 1, _TB), jnp.float32),
        ],
    )(x2d, codebook)


def _sc_gather(codebook, idx2d):
    mesh = plsc.VectorSubcoreMesh(
        core_axis_name="c", subcore_axis_name="s",
        num_cores=_NC, num_subcores=_NS)
    wpb = _TB // _BPW

    @functools.partial(
        pl.kernel,
        out_type=jax.ShapeDtypeStruct((_N, _D), jnp.float32),
        mesh=mesh,
        scratch_types=[
            pltpu.VMEM((_CPW, _CHUNK), jnp.int32),
            pltpu.VMEM((_BPW, _D), jnp.float32),
            pltpu.SemaphoreType.DMA,
        ],
    )
    def gather_kernel(table_hbm, idx_hbm, out_q, idx_v, rows_v, sem):
        wid = lax.axis_index("s") * _NC + lax.axis_index("c")
        pltpu.sync_copy(idx_hbm.at[pl.ds(wid * _CPW, _CPW)], idx_v)
        copies = [
            pltpu.async_copy(table_hbm.at[idx_v.at[j]],
                             rows_v.at[pl.ds(j * _CHUNK, _CHUNK)], sem)
            for j in range(_CPW)
        ]
        for c in copies:
            c.wait()
        pltpu.sync_copy(rows_v, out_q.at[pl.ds(wid * _BPW, _BPW)])

    return gather_kernel(codebook, idx2d)


def kernel(x, codebook):
    B, T, D = x.shape
    x2d = x.reshape(B * T, D)
    idx2d, msum = _tc_argmin(x2d, codebook)
    quant2d = _sc_gather(codebook, idx2d)
    quantized = quant2d.reshape(B, T, D)
    indices = idx2d.reshape(B, T)
    loss = 2.0 * jnp.sum(msum[0, 0, :]) / jnp.float32(B * T * D)
    return (quantized, indices, loss)
